# baseline (speedup 1.0000x reference)
import functools

import jax
import jax.numpy as jnp
from jax import lax
from jax.experimental import pallas as pl
from jax.experimental.pallas import tpu as pltpu
from jax.experimental.pallas import tpu_sc as plsc

HIDDEN = 256
N_NODES = 10000
DEG = 16
N_CLASSES = 104
HALF = HIDDEN // 2

CHUNK = 64
NBUF = 4

NPIECE = 4
NODES_P = N_NODES // NPIECE
MSGS_P = NODES_P * DEG
CHUNKS_PER_S = 40
CHUNKS_FAST = 32
CHUNKS_SLOW = 8
NCHUNKS_P = 16 * CHUNKS_PER_S
MSGS_PAD_P = NCHUNKS_P * CHUNK
IDX_PAD_CHUNKS = NCHUNKS_P + CHUNKS_FAST - CHUNKS_SLOW
ROWS_PAD_P = MSGS_PAD_P // DEG

NODES_PAD = 10240
CHUNKF = 80
NCKF = 8


@functools.cache
def _get_sc_feat():
    mesh = plsc.VectorSubcoreMesh(core_axis_name="c", subcore_axis_name="s",
                                  num_cores=1)

    @functools.partial(
        pl.kernel,
        mesh=mesh,
        out_type=jax.ShapeDtypeStruct((NODES_PAD, HIDDEN), jnp.float32),
        scratch_types=[
            pltpu.VMEM((NCKF * CHUNKF,), jnp.int32),
            pltpu.VMEM((NBUF, CHUNKF, HIDDEN), jnp.float32),
            pltpu.SemaphoreType.DMA,
            pltpu.SemaphoreType.DMA,
            pltpu.SemaphoreType.DMA,
            pltpu.SemaphoreType.DMA,
        ],
        compiler_params=pltpu.CompilerParams(needs_layout_passes=False),
    )
    def sc_feat(tok_hbm, emb_hbm, out_hbm, idxv, ring, sem0, sem1, sem2,
                sem3):
        sems = (sem0, sem1, sem2, sem3)
        base = lax.axis_index("s") * (NCKF * CHUNKF)
        pltpu.sync_copy(tok_hbm.at[pl.ds(base, NCKF * CHUNKF)], idxv)
        for b in range(NBUF):
            pltpu.async_copy(emb_hbm.at[idxv.at[pl.ds(b * CHUNKF, CHUNKF)]],
                             ring.at[b], sems[b])

        def ring_step(k, carry):
            for b in range(NBUF):
                r = NBUF * k + b
                pltpu.make_async_copy(
                    emb_hbm.at[pl.ds(0, CHUNKF)], ring.at[b], sems[b]).wait()
                pltpu.sync_copy(
                    ring.at[b],
                    out_hbm.at[pl.ds(base + r * CHUNKF, CHUNKF)])
                nr = r + NBUF

                @pl.when(nr < NCKF)
                def _():
                    pltpu.async_copy(
                        emb_hbm.at[idxv.at[pl.ds(nr * CHUNKF, CHUNKF)]],
                        ring.at[b], sems[b])
            return carry

        lax.fori_loop(0, NCKF // NBUF, ring_step, 0)

    return sc_feat


@functools.cache
def _get_sc_gather():
    mesh = plsc.VectorSubcoreMesh(core_axis_name="c", subcore_axis_name="s")

    @functools.partial(
        pl.kernel,
        mesh=mesh,
        out_type=jax.ShapeDtypeStruct((MSGS_PAD_P, HALF), jnp.int32),
        scratch_types=[
            pltpu.VMEM((CHUNKS_FAST * CHUNK,), jnp.int32),
            pltpu.VMEM((NBUF, CHUNK, HALF), jnp.int32),
            pltpu.SemaphoreType.DMA,
            pltpu.SemaphoreType.DMA,
            pltpu.SemaphoreType.DMA,
            pltpu.SemaphoreType.DMA,
        ],
        compiler_params=pltpu.CompilerParams(needs_layout_passes=False),
    )
    def sc_gather(nbr_hbm, emb_hbm, out_hbm, idx2, ring,
                  sem0, sem1, sem2, sem3):
        sems = (sem0, sem1, sem2, sem3)
        c = lax.axis_index("c")
        s = lax.axis_index("s")
        gstart = s * CHUNKS_PER_S + c * CHUNKS_FAST
        base = gstart * CHUNK
        nck = jnp.where(c == 0, CHUNKS_FAST, CHUNKS_SLOW)
        pltpu.sync_copy(nbr_hbm.at[pl.ds(base, CHUNKS_FAST * CHUNK)], idx2)

        for b in range(NBUF):
            pltpu.async_copy(emb_hbm.at[idx2.at[pl.ds(b * CHUNK, CHUNK)]],
                             ring.at[b], sems[b])

        def ring_step(k, carry):
            for b in range(NBUF):
                r = NBUF * k + b
                pltpu.make_async_copy(
                    emb_hbm.at[pl.ds(0, CHUNK)], ring.at[b], sems[b]).wait()
                pltpu.sync_copy(ring.at[b],
                                out_hbm.at[pl.ds(base + r * CHUNK, CHUNK)])
                nr = r + NBUF

                @pl.when(nr < nck)
                def _():
                    pltpu.async_copy(
                        emb_hbm.at[idx2.at[pl.ds(nr * CHUNK, CHUNK)]],
                        ring.at[b], sems[b])
            return carry

        lax.fori_loop(0, nck // NBUF, ring_step, 0)

    return sc_gather


PACK_BLOCK = 2048


def _pack_body(x_ref, o_ref):
    u = lax.bitcast_convert_type(x_ref[...], jnp.uint32)
    lo = (u[:, :HALF] + 0x7FFF + ((u[:, :HALF] >> 16) & 1)) >> 16
    hi = (u[:, HALF:] + 0x7FFF + ((u[:, HALF:] >> 16) & 1)) >> 16
    o_ref[...] = lax.bitcast_convert_type(lo | (hi << 16), jnp.int32)


def _unpack(p):
    xl = lax.bitcast_convert_type(p << 16, jnp.float32)
    xh = lax.bitcast_convert_type(p & jnp.int32(-65536), jnp.float32)
    return xl, xh


ROWS_PER_BLOCK = 512


def _gru_body(msg_ref, wih_ref, whh_ref, bih_ref, bhh_ref, lng_ref, lnb_ref,
              fcw_ref, fcb_ref, out_ref):
    wih = wih_ref[...]
    whh = whh_ref[...]
    bih = bih_ref[...]
    bhh = bhh_ref[...]
    wl = wih[:HALF, :]
    wh = wih[HALF:, :]
    hl, hh = _unpack(msg_ref[:, DEG - 1, :])
    h = jnp.concatenate([hl, hh], axis=1)
    for t in range(DEG - 1):
        xl, xh = _unpack(msg_ref[:, t, :])
        gi = (jnp.dot(xl, wl, preferred_element_type=jnp.float32)
              + jnp.dot(xh, wh, preferred_element_type=jnp.float32) + bih)
        gh = jnp.dot(h, whh, preferred_element_type=jnp.float32) + bhh
        r = jax.nn.sigmoid(gi[:, :HIDDEN] + gh[:, :HIDDEN])
        z = jax.nn.sigmoid(gi[:, HIDDEN:2 * HIDDEN] + gh[:, HIDDEN:2 * HIDDEN])
        n = jnp.tanh(gi[:, 2 * HIDDEN:] + r * gh[:, 2 * HIDDEN:])
        h = (1.0 - z) * n + z * h
    mu = jnp.mean(h, axis=-1, keepdims=True)
    var = jnp.mean((h - mu) * (h - mu), axis=-1, keepdims=True)
    ln = (h - mu) * lax.rsqrt(var + 1e-5) * lng_ref[...] + lnb_ref[...]
    out_ref[...] = jnp.dot(ln, fcw_ref[...],
                           preferred_element_type=jnp.float32) + fcb_ref[...]


_pack_call = pl.pallas_call(
    _pack_body,
    grid=(NODES_PAD // PACK_BLOCK,),
    in_specs=[pl.BlockSpec((PACK_BLOCK, HIDDEN), lambda i: (i, 0))],
    out_specs=pl.BlockSpec((PACK_BLOCK, HALF), lambda i: (i, 0)),
    out_shape=jax.ShapeDtypeStruct((NODES_PAD, HALF), jnp.int32),
    compiler_params=pltpu.CompilerParams(
        dimension_semantics=("arbitrary",),
    ),
)

_gru_call = pl.pallas_call(
    _gru_body,
    grid=(ROWS_PAD_P // ROWS_PER_BLOCK,),
    in_specs=[
        pl.BlockSpec((ROWS_PER_BLOCK, DEG, HALF), lambda i: (i, 0, 0)),
        pl.BlockSpec((HIDDEN, 3 * HIDDEN), lambda i: (0, 0)),
        pl.BlockSpec((HIDDEN, 3 * HIDDEN), lambda i: (0, 0)),
        pl.BlockSpec((1, 3 * HIDDEN), lambda i: (0, 0)),
        pl.BlockSpec((1, 3 * HIDDEN), lambda i: (0, 0)),
        pl.BlockSpec((1, HIDDEN), lambda i: (0, 0)),
        pl.BlockSpec((1, HIDDEN), lambda i: (0, 0)),
        pl.BlockSpec((HIDDEN, N_CLASSES), lambda i: (0, 0)),
        pl.BlockSpec((1, N_CLASSES), lambda i: (0, 0)),
    ],
    out_specs=pl.BlockSpec((ROWS_PER_BLOCK, N_CLASSES), lambda i: (i, 0)),
    out_shape=jax.ShapeDtypeStruct((ROWS_PAD_P, N_CLASSES), jnp.float32),
    compiler_params=pltpu.CompilerParams(
        dimension_semantics=("arbitrary",),
    ),
)


def kernel(token_ids, neighbor_idx, emb, W_ih, W_hh, b_ih, b_hh, ln_g, ln_b,
           fc_W, fc_b):
    tok = token_ids.astype(jnp.int32)
    tok = jnp.concatenate([tok, jnp.zeros((NODES_PAD - N_NODES,), jnp.int32)])
    feat = _get_sc_feat()(tok, emb)
    feat_packed = _pack_call(feat)
    sc = _get_sc_gather()
    wihT = W_ih.T
    whhT = W_hh.T
    fcT = fc_W.T
    bih = b_ih.reshape(1, -1)
    bhh = b_hh.reshape(1, -1)
    lng = ln_g.reshape(1, -1)
    lnb = ln_b.reshape(1, -1)
    fcb = fc_b.reshape(1, -1)
    outs = []
    for p in range(NPIECE):
        lo = p * ROWS_PAD_P
        hi = min(lo + ROWS_PAD_P, N_NODES)
        nbr_flat = neighbor_idx[lo:hi].reshape(-1).astype(jnp.int32)
        nbr_flat = jnp.concatenate(
            [nbr_flat,
             jnp.zeros((IDX_PAD_CHUNKS * CHUNK - (hi - lo) * DEG,),
                       jnp.int32)])
        msg_flat = sc(nbr_flat, feat_packed)
        msg = msg_flat.reshape(ROWS_PAD_P, DEG, HALF)
        outs.append(_gru_call(msg, wihT, whhT, bih, bhh, lng, lnb, fcT, fcb))
    return jnp.concatenate(outs, axis=0)[:N_NODES]


# speedup vs baseline: 1.1948x; 1.1948x over previous
import functools

import jax
import jax.numpy as jnp
from jax import lax
from jax.experimental import pallas as pl
from jax.experimental.pallas import tpu as pltpu
from jax.experimental.pallas import tpu_sc as plsc

HIDDEN = 256
N_NODES = 10000
DEG = 16
N_CLASSES = 104
HALF = HIDDEN // 2

CHUNK = 64
NBUF = 4

NPIECE = 4
NODES_P = N_NODES // NPIECE
MSGS_P = NODES_P * DEG
CHUNKS_PER_S = 40
CHUNKS_FAST = 32
CHUNKS_SLOW = 8
NCHUNKS_P = 16 * CHUNKS_PER_S
MSGS_PAD_P = NCHUNKS_P * CHUNK
IDX_PAD_CHUNKS = NCHUNKS_P + CHUNKS_FAST - CHUNKS_SLOW
ROWS_PAD_P = MSGS_PAD_P // DEG

NODES_PAD = 10240
CHUNKF = 80
NCKF = 8


@functools.cache
def _get_sc_feat():
    mesh = plsc.VectorSubcoreMesh(core_axis_name="c", subcore_axis_name="s",
                                  num_cores=1)

    @functools.partial(
        pl.kernel,
        mesh=mesh,
        out_type=jax.ShapeDtypeStruct((NODES_PAD, HIDDEN), jnp.float32),
        scratch_types=[
            pltpu.VMEM((NCKF * CHUNKF,), jnp.int32),
            pltpu.VMEM((NBUF, CHUNKF, HIDDEN), jnp.float32),
            pltpu.SemaphoreType.DMA,
            pltpu.SemaphoreType.DMA,
            pltpu.SemaphoreType.DMA,
            pltpu.SemaphoreType.DMA,
        ],
        compiler_params=pltpu.CompilerParams(needs_layout_passes=False),
    )
    def sc_feat(tok_hbm, emb_hbm, out_hbm, idxv, ring, sem0, sem1, sem2,
                sem3):
        sems = (sem0, sem1, sem2, sem3)
        base = lax.axis_index("s") * (NCKF * CHUNKF)
        pltpu.sync_copy(tok_hbm.at[pl.ds(base, NCKF * CHUNKF)], idxv)
        for b in range(NBUF):
            pltpu.async_copy(emb_hbm.at[idxv.at[pl.ds(b * CHUNKF, CHUNKF)]],
                             ring.at[b], sems[b])

        def ring_step(k, carry):
            for b in range(NBUF):
                r = NBUF * k + b
                pltpu.make_async_copy(
                    emb_hbm.at[pl.ds(0, CHUNKF)], ring.at[b], sems[b]).wait()
                pltpu.sync_copy(
                    ring.at[b],
                    out_hbm.at[pl.ds(base + r * CHUNKF, CHUNKF)])
                nr = r + NBUF

                @pl.when(nr < NCKF)
                def _():
                    pltpu.async_copy(
                        emb_hbm.at[idxv.at[pl.ds(nr * CHUNKF, CHUNKF)]],
                        ring.at[b], sems[b])
            return carry

        lax.fori_loop(0, NCKF // NBUF, ring_step, 0)

    return sc_feat


@functools.cache
def _get_sc_gather():
    mesh = plsc.VectorSubcoreMesh(core_axis_name="c", subcore_axis_name="s")

    @functools.partial(
        pl.kernel,
        mesh=mesh,
        out_type=jax.ShapeDtypeStruct((MSGS_PAD_P, HALF), jnp.int32),
        scratch_types=[
            pltpu.VMEM((CHUNKS_FAST * CHUNK,), jnp.int32),
            pltpu.VMEM((NBUF, CHUNK, HALF), jnp.int32),
            pltpu.SemaphoreType.DMA,
            pltpu.SemaphoreType.DMA,
            pltpu.SemaphoreType.DMA,
            pltpu.SemaphoreType.DMA,
        ],
        compiler_params=pltpu.CompilerParams(needs_layout_passes=False),
    )
    def sc_gather(nbr_hbm, emb_hbm, out_hbm, idx2, ring,
                  sem0, sem1, sem2, sem3):
        sems = (sem0, sem1, sem2, sem3)
        c = lax.axis_index("c")
        s = lax.axis_index("s")
        gstart = s * CHUNKS_PER_S + c * CHUNKS_FAST
        base = gstart * CHUNK
        nck = jnp.where(c == 0, CHUNKS_FAST, CHUNKS_SLOW)
        pltpu.sync_copy(nbr_hbm.at[pl.ds(base, CHUNKS_FAST * CHUNK)], idx2)

        for b in range(NBUF):
            pltpu.async_copy(emb_hbm.at[idx2.at[pl.ds(b * CHUNK, CHUNK)]],
                             ring.at[b], sems[b])

        def ring_step(k, carry):
            for b in range(NBUF):
                r = NBUF * k + b
                pltpu.make_async_copy(
                    emb_hbm.at[pl.ds(0, CHUNK)], ring.at[b], sems[b]).wait()
                pltpu.sync_copy(ring.at[b],
                                out_hbm.at[pl.ds(base + r * CHUNK, CHUNK)])
                nr = r + NBUF

                @pl.when(nr < nck)
                def _():
                    pltpu.async_copy(
                        emb_hbm.at[idx2.at[pl.ds(nr * CHUNK, CHUNK)]],
                        ring.at[b], sems[b])
            return carry

        lax.fori_loop(0, nck // NBUF, ring_step, 0)

    return sc_gather


PACK_BLOCK = 2048


def _pack_body(x_ref, o_ref):
    u = lax.bitcast_convert_type(x_ref[...], jnp.uint32)
    lo = (u[:, :HALF] + 0x7FFF + ((u[:, :HALF] >> 16) & 1)) >> 16
    hi = (u[:, HALF:] + 0x7FFF + ((u[:, HALF:] >> 16) & 1)) >> 16
    o_ref[...] = lax.bitcast_convert_type(lo | (hi << 16), jnp.int32)


def _unpack(p):
    xl = lax.bitcast_convert_type(p << 16, jnp.float32)
    xh = lax.bitcast_convert_type(p & jnp.int32(-65536), jnp.float32)
    return xl, xh


ROWS_PER_BLOCK = 512


def _gru_body(msg_ref, wih_ref, whh_ref, bih_ref, bhh_ref, lng_ref, lnb_ref,
              fcw_ref, fcb_ref, out_ref):
    wih = wih_ref[...]
    whh = whh_ref[...]
    bih = bih_ref[...]
    bhh = bhh_ref[...]
    wl = wih[:HALF, :]
    wh = wih[HALF:, :]
    hl, hh = _unpack(msg_ref[:, DEG - 1, :])
    h = jnp.concatenate([hl, hh], axis=1)
    for t in range(DEG - 1):
        xl, xh = _unpack(msg_ref[:, t, :])
        gi = (jnp.dot(xl, wl, preferred_element_type=jnp.float32)
              + jnp.dot(xh, wh, preferred_element_type=jnp.float32) + bih)
        gh = jnp.dot(h, whh, preferred_element_type=jnp.float32) + bhh
        r = jax.nn.sigmoid(gi[:, :HIDDEN] + gh[:, :HIDDEN])
        z = jax.nn.sigmoid(gi[:, HIDDEN:2 * HIDDEN] + gh[:, HIDDEN:2 * HIDDEN])
        n = jnp.tanh(gi[:, 2 * HIDDEN:] + r * gh[:, 2 * HIDDEN:])
        h = (1.0 - z) * n + z * h
    mu = jnp.mean(h, axis=-1, keepdims=True)
    var = jnp.mean((h - mu) * (h - mu), axis=-1, keepdims=True)
    ln = (h - mu) * lax.rsqrt(var + 1e-5) * lng_ref[...] + lnb_ref[...]
    out_ref[...] = jnp.dot(ln, fcw_ref[...],
                           preferred_element_type=jnp.float32) + fcb_ref[...]


_pack_call = pl.pallas_call(
    _pack_body,
    grid=(NODES_PAD // PACK_BLOCK,),
    in_specs=[pl.BlockSpec((PACK_BLOCK, HIDDEN), lambda i: (i, 0))],
    out_specs=pl.BlockSpec((PACK_BLOCK, HALF), lambda i: (i, 0)),
    out_shape=jax.ShapeDtypeStruct((NODES_PAD, HALF), jnp.int32),
    compiler_params=pltpu.CompilerParams(
        dimension_semantics=("arbitrary",),
    ),
)

_gru_call = pl.pallas_call(
    _gru_body,
    grid=(ROWS_PAD_P // ROWS_PER_BLOCK,),
    in_specs=[
        pl.BlockSpec((ROWS_PER_BLOCK, DEG, HALF), lambda i: (i, 0, 0)),
        pl.BlockSpec((HIDDEN, 3 * HIDDEN), lambda i: (0, 0)),
        pl.BlockSpec((HIDDEN, 3 * HIDDEN), lambda i: (0, 0)),
        pl.BlockSpec((1, 3 * HIDDEN), lambda i: (0, 0)),
        pl.BlockSpec((1, 3 * HIDDEN), lambda i: (0, 0)),
        pl.BlockSpec((1, HIDDEN), lambda i: (0, 0)),
        pl.BlockSpec((1, HIDDEN), lambda i: (0, 0)),
        pl.BlockSpec((HIDDEN, N_CLASSES), lambda i: (0, 0)),
        pl.BlockSpec((1, N_CLASSES), lambda i: (0, 0)),
    ],
    out_specs=pl.BlockSpec((ROWS_PER_BLOCK, N_CLASSES), lambda i: (i, 0)),
    out_shape=jax.ShapeDtypeStruct((ROWS_PAD_P, N_CLASSES), jnp.float32),
    compiler_params=pltpu.CompilerParams(
        dimension_semantics=("arbitrary",),
    ),
)


def kernel(token_ids, neighbor_idx, emb, W_ih, W_hh, b_ih, b_hh, ln_g, ln_b,
           fc_W, fc_b):
    tok = token_ids.astype(jnp.int32)
    tok = jnp.concatenate([tok, jnp.zeros((NODES_PAD - N_NODES,), jnp.int32)])
    feat = _get_sc_feat()(tok, emb)
    feat_packed = _pack_call(feat)
    sc = _get_sc_gather()
    wihT = W_ih.T
    whhT = W_hh.T
    fcT = fc_W.T
    bih = b_ih.reshape(1, -1)
    bhh = b_hh.reshape(1, -1)
    lng = ln_g.reshape(1, -1)
    lnb = ln_b.reshape(1, -1)
    fcb = fc_b.reshape(1, -1)
    outs = []
    for p in range(NPIECE):
        nbr_flat = neighbor_idx[p * NODES_P:(p + 1) * NODES_P]
        nbr_flat = nbr_flat.reshape(-1).astype(jnp.int32)
        nbr_flat = jnp.concatenate(
            [nbr_flat,
             jnp.zeros((IDX_PAD_CHUNKS * CHUNK - MSGS_P,), jnp.int32)])
        msg_flat = sc(nbr_flat, feat_packed)
        msg = msg_flat.reshape(ROWS_PAD_P, DEG, HALF)
        out_p = _gru_call(msg, wihT, whhT, bih, bhh, lng, lnb, fcT, fcb)
        outs.append(out_p[:NODES_P])
    return jnp.concatenate(outs, axis=0)
